# Initial kernel scaffold; baseline (speedup 1.0000x reference)
import jax
import jax.numpy as jnp
from jax.experimental import pallas as pl


def kernel(x, edge_index, edge_attr, batch, W0, b0, Wm, bm, Wu, bu, Wi_lstm, Wh_lstm, b_lstm, W1, b1, W2, b2):
    raise NotImplementedError("write your pallas kernel here")


# speedup vs baseline: 2.4070x; 2.4070x over previous
import functools

import jax
import jax.numpy as jnp
from jax import lax
from jax.experimental import pallas as pl
from jax.experimental.pallas import tpu as pltpu
from jax.experimental.pallas import tpu_sc as plsc

N = 10000
E = 320000
DIN = 128
DE = 16
DIM = 256
B = 64
HALF = DIM // 2
NSUB = 16
NCORE = 2
AGG_ROWS = 10240
ROWS_PER_TILE = AGG_ROWS // NSUB
E_PER_TILE = E // NSUB
EK = 80
NBATCH = E_PER_TILE // EK


def _node_pre_body(x_ref, w0_ref, b0_ref, wmt_ref, bm_ref,
                   out_ref, xm0_ref, xm1_ref):
    h = jnp.maximum(jnp.dot(x_ref[...], w0_ref[...],
                            preferred_element_type=jnp.float32) + b0_ref[...], 0.0)
    out_ref[...] = h
    xm = jnp.dot(h, wmt_ref[...], preferred_element_type=jnp.float32) + bm_ref[...]
    xm0_ref[...] = xm[:, :HALF]
    xm1_ref[...] = xm[:, HALF:]


def _node_pre(x, w0, b0, wmt, bm):
    blk = 400
    grid = (N // blk,)
    return pl.pallas_call(
        _node_pre_body,
        grid=grid,
        in_specs=[
            pl.BlockSpec((blk, DIN), lambda i: (i, 0)),
            pl.BlockSpec((DIN, DIM), lambda i: (0, 0)),
            pl.BlockSpec((1, DIM), lambda i: (0, 0)),
            pl.BlockSpec((DIM, DIM), lambda i: (0, 0)),
            pl.BlockSpec((1, DIM), lambda i: (0, 0)),
        ],
        out_specs=[
            pl.BlockSpec((blk, DIM), lambda i: (i, 0)),
            pl.BlockSpec((blk, HALF), lambda i: (i, 0)),
            pl.BlockSpec((blk, HALF), lambda i: (i, 0)),
        ],
        out_shape=[
            jax.ShapeDtypeStruct((N, DIM), jnp.float32),
            jax.ShapeDtypeStruct((N, HALF), jnp.float32),
            jax.ShapeDtypeStruct((N, HALF), jnp.float32),
        ],
    )(x, w0, b0, wmt, bm)


def _edge_pre_body(ea_ref, wmb_ref, eam0_ref, eam1_ref):
    m = jnp.dot(ea_ref[...], wmb_ref[...], preferred_element_type=jnp.float32)
    eam0_ref[...] = m[:, :HALF]
    eam1_ref[...] = m[:, HALF:]


def _edge_pre(ea, wmb):
    blk = 2000
    grid = (E // blk,)
    return pl.pallas_call(
        _edge_pre_body,
        grid=grid,
        in_specs=[
            pl.BlockSpec((blk, DE), lambda i: (i, 0)),
            pl.BlockSpec((DE, DIM), lambda i: (0, 0)),
        ],
        out_specs=[
            pl.BlockSpec((blk, HALF), lambda i: (i, 0)),
            pl.BlockSpec((blk, HALF), lambda i: (i, 0)),
        ],
        out_shape=[
            jax.ShapeDtypeStruct((E, HALF), jnp.float32),
            jax.ShapeDtypeStruct((E, HALF), jnp.float32),
        ],
    )(ea, wmb)


def _sc_body(xm0, xm1, eam0, eam1, src, dst, zeros, aggout,
             agg_sh, srcv, dstv, eamv, rowv, sem):
    c = lax.axis_index("c")
    s = lax.axis_index("s")

    pltpu.sync_copy(zeros, agg_sh.at[pl.ds(s * ROWS_PER_TILE, ROWS_PER_TILE)])
    plsc.subcore_barrier()

    ebase = s * E_PER_TILE

    def process(xm_ref, eam_ref):
        def batch_body(b, carry):
            base = ebase + b * EK
            pltpu.sync_copy(src.at[pl.ds(base, EK)], srcv)
            pltpu.sync_copy(dst.at[pl.ds(base, EK)], dstv)
            pltpu.sync_copy(eam_ref.at[pl.ds(base, EK)], eamv)
            pltpu.async_copy(xm_ref.at[srcv], rowv, sem).wait()

            def row_body(i, _):
                def col_body(k, __):
                    v = rowv[i, pl.ds(k * 16, 16)] + eamv[i, pl.ds(k * 16, 16)]
                    rowv[i, pl.ds(k * 16, 16)] = jnp.maximum(v, 0.0)
                    return 0
                return lax.fori_loop(0, HALF // 16, col_body, 0)

            lax.fori_loop(0, EK, row_body, 0)
            pltpu.sync_copy(rowv, agg_sh.at[dstv], add=True)
            return 0

        lax.fori_loop(0, NBATCH, batch_body, 0)

    @pl.when(c == 0)
    def _():
        process(xm0, eam0)

    @pl.when(c == 1)
    def _():
        process(xm1, eam1)

    plsc.subcore_barrier()

    @pl.when(c == 0)
    def _():
        pltpu.sync_copy(agg_sh.at[pl.ds(s * ROWS_PER_TILE, ROWS_PER_TILE)],
                        aggout.at[0, pl.ds(s * ROWS_PER_TILE, ROWS_PER_TILE)])

    @pl.when(c == 1)
    def _():
        pltpu.sync_copy(agg_sh.at[pl.ds(s * ROWS_PER_TILE, ROWS_PER_TILE)],
                        aggout.at[1, pl.ds(s * ROWS_PER_TILE, ROWS_PER_TILE)])


def _edge_agg(xm0, xm1, eam0, eam1, src, dst):
    zeros = jnp.zeros((ROWS_PER_TILE, HALF), jnp.float32)
    mesh = plsc.VectorSubcoreMesh(core_axis_name="c", subcore_axis_name="s")
    fn = pl.kernel(
        _sc_body,
        out_type=jax.ShapeDtypeStruct((NCORE, AGG_ROWS, HALF), jnp.float32),
        mesh=mesh,
        scratch_types=[
            pltpu.VMEM_SHARED((AGG_ROWS, HALF), jnp.float32),
            pltpu.VMEM((EK,), jnp.int32),
            pltpu.VMEM((EK,), jnp.int32),
            pltpu.VMEM((EK, HALF), jnp.float32),
            pltpu.VMEM((EK, HALF), jnp.float32),
            pltpu.SemaphoreType.DMA,
        ],
    )
    return fn(xm0, xm1, eam0, eam1, src, dst, zeros)


def _node_upd_body(out_ref, a0_ref, a1_ref, wut_ref, wua0_ref, wua1_ref, bu_ref,
                   h2_ref):
    acc = jnp.dot(out_ref[...], wut_ref[...], preferred_element_type=jnp.float32)
    acc += jnp.dot(a0_ref[...], wua0_ref[...], preferred_element_type=jnp.float32)
    acc += jnp.dot(a1_ref[...], wua1_ref[...], preferred_element_type=jnp.float32)
    h2_ref[...] = jnp.maximum(acc + bu_ref[...], 0.0)


def _node_upd(out, agg0, agg1, wut, wua0, wua1, bu):
    blk = 400
    grid = (N // blk,)
    return pl.pallas_call(
        _node_upd_body,
        grid=grid,
        in_specs=[
            pl.BlockSpec((blk, DIM), lambda i: (i, 0)),
            pl.BlockSpec((blk, HALF), lambda i: (i, 0)),
            pl.BlockSpec((blk, HALF), lambda i: (i, 0)),
            pl.BlockSpec((DIM, DIM), lambda i: (0, 0)),
            pl.BlockSpec((HALF, DIM), lambda i: (0, 0)),
            pl.BlockSpec((HALF, DIM), lambda i: (0, 0)),
            pl.BlockSpec((1, DIM), lambda i: (0, 0)),
        ],
        out_specs=pl.BlockSpec((blk, DIM), lambda i: (i, 0)),
        out_shape=jax.ShapeDtypeStruct((N, DIM), jnp.float32),
    )(out, agg0, agg1, wut, wua0, wua1, bu)


def _sig(x):
    return 1.0 / (1.0 + jnp.exp(-x))


def _s2s_body(h2_ref, batch_ref, wi_ref, wh_ref, bl_ref,
              w1_ref, b1_ref, w2_ref, b2_ref, o_ref):
    h2 = h2_ref[...]
    bcol = batch_ref[...]
    gidx = lax.broadcasted_iota(jnp.int32, (N, B), 1)
    msk = bcol == gidx
    oneh = msk.astype(jnp.float32)

    wi = wi_ref[...]
    wh = wh_ref[...]
    bl = bl_ref[...]

    hh = jnp.zeros((B, DIM), jnp.float32)
    cc = jnp.zeros((B, DIM), jnp.float32)
    qs = jnp.zeros((B, 2 * DIM), jnp.float32)
    for _ in range(3):
        gates = (jnp.dot(qs, wi, preferred_element_type=jnp.float32)
                 + jnp.dot(hh, wh, preferred_element_type=jnp.float32) + bl)
        ig = gates[:, :DIM]
        fg = gates[:, DIM:2 * DIM]
        gg = gates[:, 2 * DIM:3 * DIM]
        og = gates[:, 3 * DIM:]
        cc = _sig(fg) * cc + _sig(ig) * jnp.tanh(gg)
        hh = _sig(og) * jnp.tanh(cc)
        P = lax.dot_general(h2, hh, (((1,), (1,)), ((), ())),
                            preferred_element_type=jnp.float32)
        e = jnp.sum(jnp.where(msk, P, 0.0), axis=1, keepdims=True)
        emax = jnp.max(jnp.where(msk, e, -1e30), axis=0, keepdims=True)
        egat = jnp.sum(jnp.where(msk, emax, 0.0), axis=1, keepdims=True)
        au = jnp.exp(e - egat)
        asum = jnp.sum(jnp.where(msk, au, 0.0), axis=0, keepdims=True)
        agat = jnp.sum(jnp.where(msk, asum, 0.0), axis=1, keepdims=True)
        wa = oneh * (au / agat)
        r = lax.dot_general(wa, h2, (((0,), (0,)), ((), ())),
                            preferred_element_type=jnp.float32)
        qs = jnp.concatenate([hh, r], axis=1)

    o1 = jnp.maximum(jnp.dot(qs, w1_ref[...],
                             preferred_element_type=jnp.float32) + b1_ref[...], 0.0)
    o_ref[...] = (jnp.dot(o1, w2_ref[...], preferred_element_type=jnp.float32)
                  + b2_ref[...])


def _set2set(h2, batch2d, wi, wh, bl, w1, b1, w2, b2):
    return pl.pallas_call(
        _s2s_body,
        out_shape=jax.ShapeDtypeStruct((B, 1), jnp.float32),
    )(h2, batch2d, wi, wh, bl, w1, b1, w2, b2)


@jax.jit
def kernel(x, edge_index, edge_attr, batch, W0, b0, Wm, bm, Wu, bu,
           Wi_lstm, Wh_lstm, b_lstm, W1, b1, W2, b2):
    src = edge_index[0]
    dst = edge_index[1]

    out, xm0, xm1 = _node_pre(x, W0, b0.reshape(1, DIM), Wm[:DIM],
                              bm.reshape(1, DIM))
    eam0, eam1 = _edge_pre(edge_attr, Wm[DIM:])
    agg = _edge_agg(xm0, xm1, eam0, eam1, src, dst)
    h2 = _node_upd(out, agg[0, :N], agg[1, :N], Wu[:DIM], Wu[DIM:DIM + HALF],
                   Wu[DIM + HALF:], bu.reshape(1, DIM))
    o = _set2set(h2, batch.reshape(N, 1), Wi_lstm, Wh_lstm,
                 b_lstm.reshape(1, 4 * DIM), W1, b1.reshape(1, DIM),
                 W2, b2.reshape(1, 1))
    return o.reshape(-1)


# speedup vs baseline: 3.5528x; 1.4760x over previous
import functools

import jax
import jax.numpy as jnp
from jax import lax
from jax.experimental import pallas as pl
from jax.experimental.pallas import tpu as pltpu
from jax.experimental.pallas import tpu_sc as plsc

N = 10000
E = 320000
DIN = 128
DE = 16
DIM = 256
B = 64
HALF = DIM // 2
NSUB = 16
NCORE = 2
AGG_ROWS = 10240
ROWS_PER_TILE = AGG_ROWS // NSUB
E_PER_TILE = E // NSUB
EK = 80
NBATCH = E_PER_TILE // EK


def _node_pre_body(x_ref, w0_ref, b0_ref, wmt_ref, bm_ref,
                   out_ref, xm0_ref, xm1_ref):
    h = jnp.maximum(jnp.dot(x_ref[...], w0_ref[...],
                            preferred_element_type=jnp.float32) + b0_ref[...], 0.0)
    out_ref[...] = h
    xm = jnp.dot(h, wmt_ref[...], preferred_element_type=jnp.float32) + bm_ref[...]
    xm0_ref[...] = xm[:, :HALF]
    xm1_ref[...] = xm[:, HALF:]


def _node_pre(x, w0, b0, wmt, bm):
    blk = 400
    grid = (N // blk,)
    return pl.pallas_call(
        _node_pre_body,
        grid=grid,
        in_specs=[
            pl.BlockSpec((blk, DIN), lambda i: (i, 0)),
            pl.BlockSpec((DIN, DIM), lambda i: (0, 0)),
            pl.BlockSpec((1, DIM), lambda i: (0, 0)),
            pl.BlockSpec((DIM, DIM), lambda i: (0, 0)),
            pl.BlockSpec((1, DIM), lambda i: (0, 0)),
        ],
        out_specs=[
            pl.BlockSpec((blk, DIM), lambda i: (i, 0)),
            pl.BlockSpec((blk, HALF), lambda i: (i, 0)),
            pl.BlockSpec((blk, HALF), lambda i: (i, 0)),
        ],
        out_shape=[
            jax.ShapeDtypeStruct((N, DIM), jnp.float32),
            jax.ShapeDtypeStruct((N, HALF), jnp.float32),
            jax.ShapeDtypeStruct((N, HALF), jnp.float32),
        ],
    )(x, w0, b0, wmt, bm)


def _edge_pre_body(ea_ref, wmb_ref, eam0_ref, eam1_ref):
    m = jnp.dot(ea_ref[...], wmb_ref[...], preferred_element_type=jnp.float32)
    eam0_ref[...] = m[:, :HALF]
    eam1_ref[...] = m[:, HALF:]


def _edge_pre(ea, wmb):
    blk = 2000
    grid = (E // blk,)
    return pl.pallas_call(
        _edge_pre_body,
        grid=grid,
        in_specs=[
            pl.BlockSpec((blk, DE), lambda i: (i, 0)),
            pl.BlockSpec((DE, DIM), lambda i: (0, 0)),
        ],
        out_specs=[
            pl.BlockSpec((blk, HALF), lambda i: (i, 0)),
            pl.BlockSpec((blk, HALF), lambda i: (i, 0)),
        ],
        out_shape=[
            jax.ShapeDtypeStruct((E, HALF), jnp.float32),
            jax.ShapeDtypeStruct((E, HALF), jnp.float32),
        ],
    )(ea, wmb)


def _sc_body(xm0, xm1, eam0, eam1, src, dst, zeros, aggout,
             agg_sh,
             srcv0, srcv1, dstv0, dstv1, eamv0, eamv1, rowv0, rowv1,
             isem0, isem1, gsem0, gsem1):
    c = lax.axis_index("c")
    s = lax.axis_index("s")
    srcv = [srcv0, srcv1]
    dstv = [dstv0, dstv1]
    eamv = [eamv0, eamv1]
    rowv = [rowv0, rowv1]
    isem = [isem0, isem1]
    gsem = [gsem0, gsem1]

    pltpu.sync_copy(zeros, agg_sh.at[pl.ds(s * ROWS_PER_TILE, ROWS_PER_TILE)])
    plsc.subcore_barrier()

    ebase = s * E_PER_TILE

    def process(xm_ref, eam_ref):
        def iload(g, j):
            base = ebase + g * EK
            pltpu.async_copy(src.at[pl.ds(base, EK)], srcv[j], isem[j])
            pltpu.async_copy(dst.at[pl.ds(base, EK)], dstv[j], isem[j])
            pltpu.async_copy(eam_ref.at[pl.ds(base, EK)], eamv[j], isem[j])

        def iload_wait(g, j):
            base = ebase + g * EK
            pltpu.make_async_copy(src.at[pl.ds(base, EK)], srcv[j], isem[j]).wait()
            pltpu.make_async_copy(dst.at[pl.ds(base, EK)], dstv[j], isem[j]).wait()
            pltpu.make_async_copy(eam_ref.at[pl.ds(base, EK)], eamv[j], isem[j]).wait()

        iload(0, 0)
        iload(1, 1)
        iload_wait(0, 0)
        pltpu.async_copy(xm_ref.at[srcv[0]], rowv[0], gsem[0])

        def outer(o, carry):
            for jj in range(2):
                g = o * 2 + jj
                j, j1 = jj, 1 - jj

                @pl.when(g < NBATCH - 1)
                def _():
                    iload_wait(g + 1, j1)
                    pltpu.async_copy(xm_ref.at[srcv[j1]], rowv[j1], gsem[j1])

                pltpu.make_async_copy(xm_ref.at[srcv[j]], rowv[j], gsem[j]).wait()

                def row_body(i, _):
                    for rr in range(4):
                        for k in range(HALF // 16):
                            sl = pl.ds(k * 16, 16)
                            ii = i * 4 + rr
                            rowv[j][ii, sl] = jnp.maximum(
                                rowv[j][ii, sl] + eamv[j][ii, sl], 0.0)
                    return 0

                lax.fori_loop(0, EK // 4, row_body, 0)

                pltpu.sync_copy(rowv[j], agg_sh.at[dstv[j]], add=True)

                @pl.when(g < NBATCH - 2)
                def _():
                    iload(g + 2, j)
            return 0

        lax.fori_loop(0, NBATCH // 2, outer, 0)

    @pl.when(c == 0)
    def _():
        process(xm0, eam0)

    @pl.when(c == 1)
    def _():
        process(xm1, eam1)

    plsc.subcore_barrier()

    @pl.when(c == 0)
    def _():
        pltpu.sync_copy(agg_sh.at[pl.ds(s * ROWS_PER_TILE, ROWS_PER_TILE)],
                        aggout.at[0, pl.ds(s * ROWS_PER_TILE, ROWS_PER_TILE)])

    @pl.when(c == 1)
    def _():
        pltpu.sync_copy(agg_sh.at[pl.ds(s * ROWS_PER_TILE, ROWS_PER_TILE)],
                        aggout.at[1, pl.ds(s * ROWS_PER_TILE, ROWS_PER_TILE)])


def _edge_agg(xm0, xm1, eam0, eam1, src, dst):
    zeros = jnp.zeros((ROWS_PER_TILE, HALF), jnp.float32)
    mesh = plsc.VectorSubcoreMesh(core_axis_name="c", subcore_axis_name="s")
    fn = pl.kernel(
        _sc_body,
        out_type=jax.ShapeDtypeStruct((NCORE, AGG_ROWS, HALF), jnp.float32),
        mesh=mesh,
        scratch_types=(
            [pltpu.VMEM_SHARED((AGG_ROWS, HALF), jnp.float32)]
            + [pltpu.VMEM((EK,), jnp.int32)] * 4
            + [pltpu.VMEM((EK, HALF), jnp.float32)] * 4
            + [pltpu.SemaphoreType.DMA] * 4
        ),
    )
    return fn(xm0, xm1, eam0, eam1, src, dst, zeros)


def _node_upd_body(out_ref, a0_ref, a1_ref, wut_ref, wua0_ref, wua1_ref, bu_ref,
                   h2_ref):
    acc = jnp.dot(out_ref[...], wut_ref[...], preferred_element_type=jnp.float32)
    acc += jnp.dot(a0_ref[...], wua0_ref[...], preferred_element_type=jnp.float32)
    acc += jnp.dot(a1_ref[...], wua1_ref[...], preferred_element_type=jnp.float32)
    h2_ref[...] = jnp.maximum(acc + bu_ref[...], 0.0)


def _node_upd(out, agg0, agg1, wut, wua0, wua1, bu):
    blk = 400
    grid = (N // blk,)
    return pl.pallas_call(
        _node_upd_body,
        grid=grid,
        in_specs=[
            pl.BlockSpec((blk, DIM), lambda i: (i, 0)),
            pl.BlockSpec((blk, HALF), lambda i: (i, 0)),
            pl.BlockSpec((blk, HALF), lambda i: (i, 0)),
            pl.BlockSpec((DIM, DIM), lambda i: (0, 0)),
            pl.BlockSpec((HALF, DIM), lambda i: (0, 0)),
            pl.BlockSpec((HALF, DIM), lambda i: (0, 0)),
            pl.BlockSpec((1, DIM), lambda i: (0, 0)),
        ],
        out_specs=pl.BlockSpec((blk, DIM), lambda i: (i, 0)),
        out_shape=jax.ShapeDtypeStruct((N, DIM), jnp.float32),
    )(out, agg0, agg1, wut, wua0, wua1, bu)


def _sig(x):
    return 1.0 / (1.0 + jnp.exp(-x))


def _s2s_body(h2_ref, batch_ref, wi_ref, wh_ref, bl_ref,
              w1_ref, b1_ref, w2_ref, b2_ref, o_ref):
    h2 = h2_ref[...]
    bcol = batch_ref[...]
    gidx = lax.broadcasted_iota(jnp.int32, (N, B), 1)
    msk = bcol == gidx
    oneh = msk.astype(jnp.float32)

    wi = wi_ref[...]
    wh = wh_ref[...]
    bl = bl_ref[...]

    hh = jnp.zeros((B, DIM), jnp.float32)
    cc = jnp.zeros((B, DIM), jnp.float32)
    qs = jnp.zeros((B, 2 * DIM), jnp.float32)
    for _ in range(3):
        gates = (jnp.dot(qs, wi, preferred_element_type=jnp.float32)
                 + jnp.dot(hh, wh, preferred_element_type=jnp.float32) + bl)
        ig = gates[:, :DIM]
        fg = gates[:, DIM:2 * DIM]
        gg = gates[:, 2 * DIM:3 * DIM]
        og = gates[:, 3 * DIM:]
        cc = _sig(fg) * cc + _sig(ig) * jnp.tanh(gg)
        hh = _sig(og) * jnp.tanh(cc)
        P = lax.dot_general(h2, hh, (((1,), (1,)), ((), ())),
                            preferred_element_type=jnp.float32)
        e = jnp.sum(jnp.where(msk, P, 0.0), axis=1, keepdims=True)
        emax = jnp.max(jnp.where(msk, e, -1e30), axis=0, keepdims=True)
        egat = jnp.sum(jnp.where(msk, emax, 0.0), axis=1, keepdims=True)
        au = jnp.exp(e - egat)
        asum = jnp.sum(jnp.where(msk, au, 0.0), axis=0, keepdims=True)
        agat = jnp.sum(jnp.where(msk, asum, 0.0), axis=1, keepdims=True)
        wa = oneh * (au / agat)
        r = lax.dot_general(wa, h2, (((0,), (0,)), ((), ())),
                            preferred_element_type=jnp.float32)
        qs = jnp.concatenate([hh, r], axis=1)

    o1 = jnp.maximum(jnp.dot(qs, w1_ref[...],
                             preferred_element_type=jnp.float32) + b1_ref[...], 0.0)
    o_ref[...] = (jnp.dot(o1, w2_ref[...], preferred_element_type=jnp.float32)
                  + b2_ref[...])


def _set2set(h2, batch2d, wi, wh, bl, w1, b1, w2, b2):
    return pl.pallas_call(
        _s2s_body,
        out_shape=jax.ShapeDtypeStruct((B, 1), jnp.float32),
    )(h2, batch2d, wi, wh, bl, w1, b1, w2, b2)


@jax.jit
def kernel(x, edge_index, edge_attr, batch, W0, b0, Wm, bm, Wu, bu,
           Wi_lstm, Wh_lstm, b_lstm, W1, b1, W2, b2):
    src = edge_index[0]
    dst = edge_index[1]

    out, xm0, xm1 = _node_pre(x, W0, b0.reshape(1, DIM), Wm[:DIM],
                              bm.reshape(1, DIM))
    eam0, eam1 = _edge_pre(edge_attr, Wm[DIM:])
    agg = _edge_agg(xm0, xm1, eam0, eam1, src, dst)
    h2 = _node_upd(out, agg[0, :N], agg[1, :N], Wu[:DIM], Wu[DIM:DIM + HALF],
                   Wu[DIM + HALF:], bu.reshape(1, DIM))
    o = _set2set(h2, batch.reshape(N, 1), Wi_lstm, Wh_lstm,
                 b_lstm.reshape(1, 4 * DIM), W1, b1.reshape(1, DIM),
                 W2, b2.reshape(1, 1))
    return o.reshape(-1)
